# baseline (speedup 1.0000x reference)
import jax
import jax.numpy as jnp
from jax.experimental import pallas as pl
from jax.experimental.pallas import tpu as pltpu

_BLOCK_M = 8192
_N_CODES = 1024
_DIM = 64
_SCALE = float(2 ** 23)
_MAGIC = float(1.5 * 2 ** 33)
_B0 = (160 << 23) | (1 << 22)
_K0 = (_B0 * 1024) % (2 ** 32)
_K0 = _K0 - 2 ** 32 if _K0 >= 2 ** 31 else _K0
_T0 = 2 ** 27


def _vq_body(xt_ref, w_ref, o_ref):
    w = w_ref[...]
    xt = xt_ref[...]
    dots_s = jax.lax.dot_general(
        w, xt, (((1,), (0,)), ((), ())),
        preferred_element_type=jnp.float32)
    wsq = jnp.sum(w * w, axis=1, keepdims=True)
    mj = jnp.float32(_MAGIC) - wsq * jnp.float32(0.5 * _SCALE)
    row = jax.lax.broadcasted_iota(jnp.int32, wsq.shape, 0)
    c = (2 ** 30 + 1023 - _K0 + _T0) - row
    g = dots_s + mj
    u = jax.lax.bitcast_convert_type(g, jnp.int32)
    key = (u << 10) + c
    kf = jax.lax.bitcast_convert_type(key, jnp.float32)
    m = jnp.max(kf, axis=0, keepdims=True)
    mi = jax.lax.bitcast_convert_type(m, jnp.int32)
    idx = 1023 - (mi & 1023)
    o_ref[...] = idx[None]


def kernel(x, W):
    n = x.shape[0]
    grid = n // _BLOCK_M
    xt = (x * jnp.float32(_SCALE)).T
    out = pl.pallas_call(
        _vq_body,
        grid=(grid,),
        in_specs=[
            pl.BlockSpec((_DIM, _BLOCK_M), lambda i: (0, i)),
            pl.BlockSpec((_N_CODES, _DIM), lambda i: (0, 0)),
        ],
        out_specs=pl.BlockSpec((1, 1, _BLOCK_M), lambda i: (i, 0, 0)),
        out_shape=jax.ShapeDtypeStruct((grid, 1, _BLOCK_M), jnp.int32),
        compiler_params=pltpu.CompilerParams(
            dimension_semantics=("arbitrary",)),
    )(xt, W)
    return out.reshape(n)


# speedup vs baseline: 1.2409x; 1.2409x over previous
import jax
import jax.numpy as jnp
from jax.experimental import pallas as pl
from jax.experimental.pallas import tpu as pltpu

_BLOCK_M = 8192
_N_CODES = 1024
_DIM = 64
_SCALE = float(2 ** 23)
_MAGIC = float(1.5 * 2 ** 33)
_B0 = (160 << 23) | (1 << 22)
_K0 = (_B0 * 1024) % (2 ** 32)
_K0 = _K0 - 2 ** 32 if _K0 >= 2 ** 31 else _K0
_T0 = 2 ** 27


def _vq_body(xt_ref, w_ref, o_ref):
    w = w_ref[...]
    xt = xt_ref[...]
    xs = xt * jnp.float32(_SCALE)
    dots_s = jax.lax.dot_general(
        w, xs, (((1,), (0,)), ((), ())),
        preferred_element_type=jnp.float32)
    wsq = jnp.sum(w * w, axis=1, keepdims=True)
    mj = jnp.float32(_MAGIC) - wsq * jnp.float32(0.5 * _SCALE)
    row = jax.lax.broadcasted_iota(jnp.int32, wsq.shape, 0)
    c = (2 ** 30 + 1023 - _K0 + _T0) - row
    g = dots_s + mj
    u = jax.lax.bitcast_convert_type(g, jnp.int32)
    key = (u << 10) + c
    kf = jax.lax.bitcast_convert_type(key, jnp.float32)
    m = jnp.max(kf, axis=0, keepdims=True)
    mi = jax.lax.bitcast_convert_type(m, jnp.int32)
    idx = 1023 - (mi & 1023)
    o_ref[...] = idx[None]


def kernel(x, W):
    n = x.shape[0]
    grid = n // _BLOCK_M
    xt = x.T
    out = pl.pallas_call(
        _vq_body,
        grid=(grid,),
        in_specs=[
            pl.BlockSpec((_DIM, _BLOCK_M), lambda i: (0, i)),
            pl.BlockSpec((_N_CODES, _DIM), lambda i: (0, 0)),
        ],
        out_specs=pl.BlockSpec((1, 1, _BLOCK_M), lambda i: (i, 0, 0)),
        out_shape=jax.ShapeDtypeStruct((grid, 1, _BLOCK_M), jnp.int32),
        compiler_params=pltpu.CompilerParams(
            dimension_semantics=("arbitrary",)),
    )(xt, W)
    return out.reshape(n)
